# baseline (speedup 1.0000x reference)
import functools

import jax
import jax.numpy as jnp
from jax import lax
from jax.experimental import pallas as pl
from jax.experimental.pallas import tpu as pltpu
from jax.experimental.pallas import tpu_sc as plsc

B, S, D = 16384, 100, 64
L = 16
NC, NS = 2, 16
NW = NC * NS
PER_W = B // NW
NB = 8
NPAIR = PER_W // (2 * NB)

_mesh = plsc.VectorSubcoreMesh(
    core_axis_name="c", subcore_axis_name="s", num_cores=NC, num_subcores=NS
)


_GATHER_DNUMS = lax.GatherDimensionNumbers(
    offset_dims=(), collapsed_slice_dims=(0,), start_index_map=(0,)
)


def _lane_shuffle(p, perm):
    return lax.gather(
        p,
        perm[:, None],
        _GATHER_DNUMS,
        slice_sizes=(1,),
        mode=lax.GatherScatterMode.PROMISE_IN_BOUNDS,
    )


def _xlane_sum(p, lanes):
    for sh in (8, 4, 2, 1):
        p = p + _lane_shuffle(p, lanes ^ sh)
    return p


def _process_chunk(buf, outv, boff, idxacc):
    lanes = lax.iota(jnp.int32, L)
    for b in range(NB):
        def jbody(j, carry):
            bestv, besti, r0, r1, r2, r3 = carry
            v0 = buf[b, j, pl.ds(0, L)]
            v1 = buf[b, j, pl.ds(L, L)]
            v2 = buf[b, j, pl.ds(2 * L, L)]
            v3 = buf[b, j, pl.ds(3 * L, L)]
            p = v0 * v0 + v1 * v1 + v2 * v2 + v3 * v3
            nrm = _xlane_sum(p, lanes)
            jv = jnp.full((L,), j, jnp.int32)
            better = nrm > bestv
            bestv = jnp.where(better, nrm, bestv)
            besti = jnp.where(better, jv, besti)
            r0 = jnp.where(better, v0, r0)
            r1 = jnp.where(better, v1, r1)
            r2 = jnp.where(better, v2, r2)
            r3 = jnp.where(better, v3, r3)
            return bestv, besti, r0, r1, r2, r3

        init = (
            jnp.full((L,), -1.0, jnp.float32),
            jnp.zeros((L,), jnp.int32),
            jnp.zeros((L,), jnp.float32),
            jnp.zeros((L,), jnp.float32),
            jnp.zeros((L,), jnp.float32),
            jnp.zeros((L,), jnp.float32),
        )
        _, besti, r0, r1, r2, r3 = lax.fori_loop(0, S, jbody, init)
        outv[boff + b, pl.ds(0, L)] = r0
        outv[boff + b, pl.ds(L, L)] = r1
        outv[boff + b, pl.ds(2 * L, L)] = r2
        outv[boff + b, pl.ds(3 * L, L)] = r3
        idxacc = jnp.where(lanes == (boff + b), besti, idxacc)
    return idxacc


@functools.partial(
    pl.kernel,
    out_type=(
        jax.ShapeDtypeStruct((B, D), jnp.float32),
        jax.ShapeDtypeStruct((B,), jnp.int32),
    ),
    mesh=_mesh,
    scratch_types=[
        pltpu.VMEM((NB, S, D), jnp.float32),
        pltpu.VMEM((NB, S, D), jnp.float32),
        pltpu.VMEM((2 * NB, D), jnp.float32),
        pltpu.VMEM((PER_W,), jnp.int32),
        pltpu.SemaphoreType.DMA,
        pltpu.SemaphoreType.DMA,
    ],
    compiler_params=pltpu.CompilerParams(use_tc_tiling_on_sc=False),
)
def _mask_cid(x_hbm, out_hbm, idx_hbm, buf0, buf1, outv, idxfull, sem0, sem1):
    wid = lax.axis_index("s") * NC + lax.axis_index("c")
    base0 = wid * PER_W

    def start(g, buf, sem):
        pltpu.async_copy(x_hbm.at[pl.ds(base0 + g * NB, NB)], buf, sem)

    def wait(g, buf, sem):
        pltpu.make_async_copy(
            x_hbm.at[pl.ds(base0 + g * NB, NB)], buf, sem
        ).wait()

    start(0, buf0, sem0)

    def pair_body(gp, _):
        g0 = 2 * gp
        wait(g0, buf0, sem0)
        start(g0 + 1, buf1, sem1)
        idxacc = jnp.zeros((L,), jnp.int32)
        idxacc = _process_chunk(buf0, outv, 0, idxacc)
        wait(g0 + 1, buf1, sem1)

        @pl.when(gp + 1 < NPAIR)
        def _():
            start(g0 + 2, buf0, sem0)

        idxacc = _process_chunk(buf1, outv, NB, idxacc)
        idxfull[pl.ds(pl.multiple_of(gp * 2 * NB, L), L)] = idxacc
        pltpu.sync_copy(outv, out_hbm.at[pl.ds(base0 + g0 * NB, 2 * NB)])
        return 0

    lax.fori_loop(0, NPAIR, pair_body, 0)
    pltpu.sync_copy(idxfull, idx_hbm.at[pl.ds(base0, PER_W)])


def kernel(x):
    out2d, idx = _mask_cid(x)
    return out2d.reshape(B, 1, D), idx
